# baseline (speedup 1.0000x reference)
import jax
import jax.numpy as jnp
import numpy as np
from jax import lax
from jax.experimental import pallas as pl
from jax.experimental.pallas import tpu as pltpu
from jax.experimental.pallas import tpu_sc as plsc

VOCAB = 100000
DIM = 128
B = 1024
L = 200
LP = 256
NLV = 13
NC = 2
NS = 16
NW = NC * NS
ROWS_PER_W = B // NW
HA = 96
HB = 104
GA_N = HA // 16
GB_N = HB // 16

PI = np.float32(np.pi)
QS = tuple(np.float32(v) for v in
           (-0.9994502067565918, 0.16583843529224396,
            -0.007998578250408173, 0.00014774066221434623))
RC = tuple(np.float32(v) for v in
           (-0.9989871382713318, 0.49624863266944885,
            -0.03952230140566826, 0.0009928615763783455))


def _poly(u, coeffs):
    acc = coeffs[-1]
    for c in reversed(coeffs[:-1]):
        acc = acc * u + c
    return acc


def _rsqrt_nr(s):
    i = plsc.bitcast(s, jnp.int32)
    i = jnp.int32(0x5F3759DF) - lax.shift_right_logical(i, 1)
    y = plsc.bitcast(i, jnp.float32)
    hs = s * np.float32(0.5)
    for _ in range(3):
        y = y * (np.float32(1.5) - hs * y * y)
    return y


def _drain(src, dst, sem):
    pltpu.make_async_copy(src, dst, sem).wait()


def _body(doc_hbm, amp_hbm, ph_hbm, real_hbm, imag_hbm, w_hbm,
          idx_a, idx_b, ga0, gp0, ga1, gp1, ga2, gp2, ga3, gp3, wbuf,
          gsem0, gsem1, gsem2, gsem3, osem0, osem1, osem2, osem3):
    wid = lax.axis_index("s") * NC + lax.axis_index("c")
    base = wid * ROWS_PER_W
    lane = lax.broadcasted_iota(jnp.int32, (16,), 0)

    sets = ((ga0, gp0, gsem0, osem0), (ga1, gp1, gsem1, osem1),
            (ga2, gp2, gsem2, osem2), (ga3, gp3, gsem3, osem3))
    spans = (pl.ds(0, HA), pl.ds(HA, HB))

    for i in range(ROWS_PER_W):
        r = (base + i) * L
        pltpu.async_copy(doc_hbm.at[pl.ds(r, HA)], idx_a.at[i, 0], osem0)
        pltpu.async_copy(doc_hbm.at[pl.ds(r + HA, HB)], idx_b.at[i, 0], osem0)
    for i in range(ROWS_PER_W):
        _drain(doc_hbm.at[pl.ds(0, HA)], idx_a.at[i, 0], osem0)
        _drain(doc_hbm.at[pl.ds(0, HB)], idx_b.at[i, 0], osem0)

    def fire_gathers(i, h, ga, gp, gsem):
        idx = (idx_a if h == 0 else idx_b).at[i, 0]
        pltpu.async_copy(amp_hbm.at[idx], ga, gsem)
        pltpu.async_copy(ph_hbm.at[idx], gp, gsem)

    def one_token(ga, gp, lt):
        acc = jnp.zeros((16,), jnp.float32)
        for j in range(DIM // 16):
            sl = pl.ds(j * 16, 16)
            a = ga[lt, sl]
            p = gp[lt, sl]
            tt = p - PI
            u = tt * tt
            ga[lt, sl] = a * _poly(u, RC)
            gp[lt, sl] = (a * tt) * _poly(u, QS)
            acc = acc + a * a
        return jnp.sum(acc)

    def do_half(ga, gp, i, h):
        goff = h * GA_N

        def group(lt0, n_tok, gidx, init):
            @plsc.parallel_loop(0, n_tok, unroll=2, carry=init)
            def g(ti, gacc):
                return jnp.where(lane == ti, one_token(ga, gp, lt0 + ti), gacc)
            wbuf[i, pl.ds(gidx * 16, 16)] = g

        @plsc.parallel_loop(0, GA_N if h == 0 else GB_N)
        def _(lg):
            group(lg * 16, 16, goff + lg, jnp.zeros((16,), jnp.float32))
        if h == 1:
            group(GB_N * 16, HB - GB_N * 16, goff + GB_N,
                  jnp.full((16,), 1e-30, jnp.float32))

    def softmax_row(i):
        svs = [wbuf[i, pl.ds(k * 16, 16)] for k in range(NLV)]
        nvs = [s * _rsqrt_nr(s) for s in svs]
        m = nvs[0]
        for v in nvs[1:]:
            m = jnp.maximum(m, v)
        mm = jnp.max(m)
        evs = [jnp.exp(v - mm) for v in nvs]
        tot = evs[0]
        for v in evs[1:]:
            tot = tot + v
        rt = _rsqrt_nr(jnp.broadcast_to(jnp.sum(tot), (16,)))
        inv = rt * rt
        for k in range(NLV):
            wbuf[i, pl.ds(k * 16, 16)] = evs[k] * inv

    def task(t_off, k, drain_prev_cond, fire_next_cond):
        h = t_off % 2
        i = 2 * k + t_off // 2
        ga, gp, gsem, osem = sets[t_off]
        nga, ngp, ngsem, nosem = sets[(t_off + 2) % 4]
        span = spans[h]
        _drain(amp_hbm.at[span], ga, gsem)
        _drain(amp_hbm.at[span], gp, gsem)
        do_half(ga, gp, i, h)
        if h == 1:
            softmax_row(i)
        def drain_prev():
            _drain(nga, real_hbm.at[0, span], nosem)
            _drain(ngp, imag_hbm.at[0, span], nosem)

        def fire_next():
            fire_gathers(i + 1, h, nga, ngp, ngsem)

        if drain_prev_cond is True:
            drain_prev()
        else:
            pl.when(drain_prev_cond)(drain_prev)
        if fire_next_cond is True:
            fire_next()
        else:
            pl.when(fire_next_cond)(fire_next)
        pltpu.async_copy(ga, real_hbm.at[base + i, span], osem)
        pltpu.async_copy(gp, imag_hbm.at[base + i, span], osem)

    fire_gathers(0, 0, ga0, gp0, gsem0)
    fire_gathers(0, 1, ga1, gp1, gsem1)

    def quad_body(k, c):
        task(0, k, k > 0, True)
        task(1, k, k > 0, True)
        task(2, k, True, k < ROWS_PER_W // 2 - 1)
        task(3, k, True, k < ROWS_PER_W // 2 - 1)
        return c
    lax.fori_loop(0, ROWS_PER_W // 2, quad_body, 0)

    _drain(ga2, real_hbm.at[0, spans[0]], osem2)
    _drain(gp2, imag_hbm.at[0, spans[0]], osem2)
    _drain(ga3, real_hbm.at[0, spans[1]], osem3)
    _drain(gp3, imag_hbm.at[0, spans[1]], osem3)
    pltpu.sync_copy(wbuf, w_hbm.at[pl.ds(base, ROWS_PER_W)])


_sc_call = pl.kernel(
    _body,
    out_type=(
        jax.ShapeDtypeStruct((B, L, DIM), jnp.float32),
        jax.ShapeDtypeStruct((B, L, DIM), jnp.float32),
        jax.ShapeDtypeStruct((B, LP), jnp.float32),
    ),
    mesh=plsc.VectorSubcoreMesh(core_axis_name="c", subcore_axis_name="s"),
    scratch_types=[
        pltpu.VMEM((ROWS_PER_W, 1, HA), jnp.int32),
        pltpu.VMEM((ROWS_PER_W, 1, HB), jnp.int32),
        pltpu.VMEM((HA, DIM), jnp.float32),
        pltpu.VMEM((HA, DIM), jnp.float32),
        pltpu.VMEM((HB, DIM), jnp.float32),
        pltpu.VMEM((HB, DIM), jnp.float32),
        pltpu.VMEM((HA, DIM), jnp.float32),
        pltpu.VMEM((HA, DIM), jnp.float32),
        pltpu.VMEM((HB, DIM), jnp.float32),
        pltpu.VMEM((HB, DIM), jnp.float32),
        pltpu.VMEM((ROWS_PER_W, LP), jnp.float32),
        pltpu.SemaphoreType.DMA,
        pltpu.SemaphoreType.DMA,
        pltpu.SemaphoreType.DMA,
        pltpu.SemaphoreType.DMA,
        pltpu.SemaphoreType.DMA,
        pltpu.SemaphoreType.DMA,
        pltpu.SemaphoreType.DMA,
        pltpu.SemaphoreType.DMA,
    ],
    compiler_params=pltpu.CompilerParams(needs_layout_passes=False),
)


def kernel(doc, amplitude_table, phase_table):
    doc_flat = doc.reshape(B * L).astype(jnp.int32)
    real, imag, w = _sc_call(doc_flat, amplitude_table, phase_table)
    return real, imag, w[:, :L]


# speedup vs baseline: 1.1271x; 1.1271x over previous
import jax
import jax.numpy as jnp
import numpy as np
from jax import lax
from jax.experimental import pallas as pl
from jax.experimental.pallas import tpu as pltpu
from jax.experimental.pallas import tpu_sc as plsc

VOCAB = 100000
DIM = 128
B = 1024
L = 200
LP = 256
NLV = 13
NC = 2
NS = 16
NW = NC * NS
ROWS_PER_W = B // NW
HA = 96
HB = 104
GA_N = HA // 16
GB_N = HB // 16

PI = np.float32(np.pi)
QS = tuple(np.float32(v) for v in
           (-0.9878619909286499, 0.15527130663394928, -0.005643106531351805))
RC = tuple(np.float32(v) for v in
           (-0.9989871382713318, 0.49624863266944885,
            -0.03952230140566826, 0.0009928615763783455))


def _poly(u, coeffs):
    acc = coeffs[-1]
    for c in reversed(coeffs[:-1]):
        acc = acc * u + c
    return acc


def _rsqrt_nr(s):
    i = plsc.bitcast(s, jnp.int32)
    i = jnp.int32(0x5F3759DF) - lax.shift_right_logical(i, 1)
    y = plsc.bitcast(i, jnp.float32)
    hs = s * np.float32(0.5)
    for _ in range(3):
        y = y * (np.float32(1.5) - hs * y * y)
    return y


def _drain(src, dst, sem):
    pltpu.make_async_copy(src, dst, sem).wait()


def _body(doc_hbm, amp_hbm, ph_hbm, real_hbm, imag_hbm, w_hbm,
          idx_a, idx_b, ga0, gp0, ga1, gp1, ga2, gp2, ga3, gp3, wbuf,
          gsem0, gsem1, gsem2, gsem3, osem0, osem1, osem2, osem3):
    wid = lax.axis_index("s") * NC + lax.axis_index("c")
    base = wid * ROWS_PER_W
    lane = lax.broadcasted_iota(jnp.int32, (16,), 0)

    sets = ((ga0, gp0, gsem0, osem0), (ga1, gp1, gsem1, osem1),
            (ga2, gp2, gsem2, osem2), (ga3, gp3, gsem3, osem3))
    spans = (pl.ds(0, HA), pl.ds(HA, HB))

    for i in range(ROWS_PER_W):
        r = (base + i) * L
        pltpu.async_copy(doc_hbm.at[pl.ds(r, HA)], idx_a.at[i, 0], osem0)
        pltpu.async_copy(doc_hbm.at[pl.ds(r + HA, HB)], idx_b.at[i, 0], osem0)
    for i in range(ROWS_PER_W):
        _drain(doc_hbm.at[pl.ds(0, HA)], idx_a.at[i, 0], osem0)
        _drain(doc_hbm.at[pl.ds(0, HB)], idx_b.at[i, 0], osem0)

    def fire_gathers(i, h, ga, gp, gsem):
        idx = (idx_a if h == 0 else idx_b).at[i, 0]
        pltpu.async_copy(amp_hbm.at[idx], ga, gsem)
        pltpu.async_copy(ph_hbm.at[idx], gp, gsem)

    def one_token(ga, gp, lt):
        acc = jnp.zeros((16,), jnp.float32)
        for j in range(DIM // 16):
            sl = pl.ds(j * 16, 16)
            a = ga[lt, sl]
            p = gp[lt, sl]
            tt = p - PI
            u = tt * tt
            ga[lt, sl] = a * _poly(u, RC)
            gp[lt, sl] = (a * tt) * _poly(u, QS)
            acc = acc + a * a
        return jnp.sum(acc)

    def do_half(ga, gp, i, h):
        goff = h * GA_N

        def group(lt0, n_tok, gidx, init):
            @plsc.parallel_loop(0, n_tok, unroll=4, carry=init)
            def g(ti, gacc):
                return jnp.where(lane == ti, one_token(ga, gp, lt0 + ti), gacc)
            wbuf[i, pl.ds(gidx * 16, 16)] = g

        @plsc.parallel_loop(0, GA_N if h == 0 else GB_N)
        def _(lg):
            group(lg * 16, 16, goff + lg, jnp.zeros((16,), jnp.float32))
        if h == 1:
            group(GB_N * 16, HB - GB_N * 16, goff + GB_N,
                  jnp.full((16,), 1e-30, jnp.float32))

    def softmax_row(i):
        svs = [wbuf[i, pl.ds(k * 16, 16)] for k in range(NLV)]
        nvs = [s * _rsqrt_nr(s) for s in svs]
        m = nvs[0]
        for v in nvs[1:]:
            m = jnp.maximum(m, v)
        mm = jnp.max(m)
        evs = [jnp.exp(v - mm) for v in nvs]
        tot = evs[0]
        for v in evs[1:]:
            tot = tot + v
        rt = _rsqrt_nr(jnp.broadcast_to(jnp.sum(tot), (16,)))
        inv = rt * rt
        for k in range(NLV):
            wbuf[i, pl.ds(k * 16, 16)] = evs[k] * inv

    def task(t_off, k, drain_prev_cond, fire_next_cond):
        h = t_off % 2
        i = 2 * k + t_off // 2
        ga, gp, gsem, osem = sets[t_off]
        nga, ngp, ngsem, nosem = sets[(t_off + 2) % 4]
        span = spans[h]
        _drain(amp_hbm.at[span], ga, gsem)
        _drain(amp_hbm.at[span], gp, gsem)
        do_half(ga, gp, i, h)
        if h == 1:
            softmax_row(i)
        def drain_prev():
            _drain(nga, real_hbm.at[0, span], nosem)
            _drain(ngp, imag_hbm.at[0, span], nosem)

        def fire_next():
            fire_gathers(i + 1, h, nga, ngp, ngsem)

        if drain_prev_cond is True:
            drain_prev()
        else:
            pl.when(drain_prev_cond)(drain_prev)
        if fire_next_cond is True:
            fire_next()
        else:
            pl.when(fire_next_cond)(fire_next)
        pltpu.async_copy(ga, real_hbm.at[base + i, span], osem)
        pltpu.async_copy(gp, imag_hbm.at[base + i, span], osem)

    fire_gathers(0, 0, ga0, gp0, gsem0)
    fire_gathers(0, 1, ga1, gp1, gsem1)

    def quad_body(k, c):
        task(0, k, k > 0, True)
        task(1, k, k > 0, True)
        task(2, k, True, k < ROWS_PER_W // 2 - 1)
        task(3, k, True, k < ROWS_PER_W // 2 - 1)
        return c
    lax.fori_loop(0, ROWS_PER_W // 2, quad_body, 0)

    _drain(ga2, real_hbm.at[0, spans[0]], osem2)
    _drain(gp2, imag_hbm.at[0, spans[0]], osem2)
    _drain(ga3, real_hbm.at[0, spans[1]], osem3)
    _drain(gp3, imag_hbm.at[0, spans[1]], osem3)
    pltpu.sync_copy(wbuf, w_hbm.at[pl.ds(base, ROWS_PER_W)])


_sc_call = pl.kernel(
    _body,
    out_type=(
        jax.ShapeDtypeStruct((B, L, DIM), jnp.float32),
        jax.ShapeDtypeStruct((B, L, DIM), jnp.float32),
        jax.ShapeDtypeStruct((B, LP), jnp.float32),
    ),
    mesh=plsc.VectorSubcoreMesh(core_axis_name="c", subcore_axis_name="s"),
    scratch_types=[
        pltpu.VMEM((ROWS_PER_W, 1, HA), jnp.int32),
        pltpu.VMEM((ROWS_PER_W, 1, HB), jnp.int32),
        pltpu.VMEM((HA, DIM), jnp.float32),
        pltpu.VMEM((HA, DIM), jnp.float32),
        pltpu.VMEM((HB, DIM), jnp.float32),
        pltpu.VMEM((HB, DIM), jnp.float32),
        pltpu.VMEM((HA, DIM), jnp.float32),
        pltpu.VMEM((HA, DIM), jnp.float32),
        pltpu.VMEM((HB, DIM), jnp.float32),
        pltpu.VMEM((HB, DIM), jnp.float32),
        pltpu.VMEM((ROWS_PER_W, LP), jnp.float32),
        pltpu.SemaphoreType.DMA,
        pltpu.SemaphoreType.DMA,
        pltpu.SemaphoreType.DMA,
        pltpu.SemaphoreType.DMA,
        pltpu.SemaphoreType.DMA,
        pltpu.SemaphoreType.DMA,
        pltpu.SemaphoreType.DMA,
        pltpu.SemaphoreType.DMA,
    ],
    compiler_params=pltpu.CompilerParams(needs_layout_passes=False),
)


def kernel(doc, amplitude_table, phase_table):
    doc_flat = doc.reshape(B * L).astype(jnp.int32)
    real, imag, w = _sc_call(doc_flat, amplitude_table, phase_table)
    return real, imag, w[:, :L]


# speedup vs baseline: 1.1323x; 1.0046x over previous
import jax
import jax.numpy as jnp
import numpy as np
from jax import lax
from jax.experimental import pallas as pl
from jax.experimental.pallas import tpu as pltpu
from jax.experimental.pallas import tpu_sc as plsc

VOCAB = 100000
DIM = 128
B = 1024
L = 200
LP = 256
NLV = 13
NC = 2
NS = 16
NW = NC * NS
ROWS_PER_W = B // NW
HA = 96
HB = 104
GA_N = HA // 16
GB_N = HB // 16

PI = np.float32(np.pi)
QS = tuple(np.float32(v) for v in
           (-0.9878619909286499, 0.15527130663394928, -0.005643106531351805))
RC = tuple(np.float32(v) for v in
           (-0.9989871382713318, 0.49624863266944885,
            -0.03952230140566826, 0.0009928615763783455))


def _poly(u, coeffs):
    acc = coeffs[-1]
    for c in reversed(coeffs[:-1]):
        acc = acc * u + c
    return acc


def _rsqrt_nr(s):
    i = plsc.bitcast(s, jnp.int32)
    i = jnp.int32(0x5F3759DF) - lax.shift_right_logical(i, 1)
    y = plsc.bitcast(i, jnp.float32)
    hs = s * np.float32(0.5)
    for _ in range(3):
        y = y * (np.float32(1.5) - hs * y * y)
    return y


def _drain(src, dst, sem):
    pltpu.make_async_copy(src, dst, sem).wait()


def _body(doc_hbm, amp_hbm, ph_hbm, real_hbm, imag_hbm, w_hbm,
          idx_a, idx_b, ga0, gp0, ga1, gp1, ga2, gp2, ga3, gp3, wbuf,
          gsem0, gsem1, gsem2, gsem3, osem0, osem1, osem2, osem3):
    wid = lax.axis_index("s") * NC + lax.axis_index("c")
    base = wid * ROWS_PER_W
    lane = lax.broadcasted_iota(jnp.int32, (16,), 0)

    sets = ((ga0, gp0, gsem0, osem0), (ga1, gp1, gsem1, osem1),
            (ga2, gp2, gsem2, osem2), (ga3, gp3, gsem3, osem3))
    spans = (pl.ds(0, HA), pl.ds(HA, HB))

    for i in range(ROWS_PER_W):
        r = (base + i) * L
        pltpu.async_copy(doc_hbm.at[pl.ds(r, HA)], idx_a.at[i, 0], osem0)
        pltpu.async_copy(doc_hbm.at[pl.ds(r + HA, HB)], idx_b.at[i, 0], osem0)
    for i in range(ROWS_PER_W):
        _drain(doc_hbm.at[pl.ds(0, HA)], idx_a.at[i, 0], osem0)
        _drain(doc_hbm.at[pl.ds(0, HB)], idx_b.at[i, 0], osem0)

    def fire_gathers(i, h, ga, gp, gsem):
        idx = (idx_a if h == 0 else idx_b).at[i, 0]
        pltpu.async_copy(amp_hbm.at[idx], ga, gsem)
        pltpu.async_copy(ph_hbm.at[idx], gp, gsem)

    def one_token(ga, gp, lt):
        acc = jnp.zeros((16,), jnp.float32)
        for j in range(DIM // 16):
            sl = pl.ds(j * 16, 16)
            a = ga[lt, sl]
            p = gp[lt, sl]
            tt = p - PI
            u = tt * tt
            ga[lt, sl] = a * _poly(u, RC)
            gp[lt, sl] = (a * tt) * _poly(u, QS)
            acc = acc + a * a
        return jnp.sum(acc)

    def do_half(ga, gp, i, h):
        goff = h * GA_N

        def group(lt0, n_tok, gidx, init):
            @plsc.parallel_loop(0, n_tok, unroll=8, carry=init)
            def g(ti, gacc):
                return jnp.where(lane == ti, one_token(ga, gp, lt0 + ti), gacc)
            wbuf[i, pl.ds(gidx * 16, 16)] = g

        @plsc.parallel_loop(0, GA_N if h == 0 else GB_N)
        def _(lg):
            group(lg * 16, 16, goff + lg, jnp.zeros((16,), jnp.float32))
        if h == 1:
            group(GB_N * 16, HB - GB_N * 16, goff + GB_N,
                  jnp.full((16,), 1e-30, jnp.float32))

    def softmax_row(i):
        svs = [wbuf[i, pl.ds(k * 16, 16)] for k in range(NLV)]
        nvs = [s * _rsqrt_nr(s) for s in svs]
        m = nvs[0]
        for v in nvs[1:]:
            m = jnp.maximum(m, v)
        mm = jnp.max(m)
        evs = [jnp.exp(v - mm) for v in nvs]
        tot = evs[0]
        for v in evs[1:]:
            tot = tot + v
        rt = _rsqrt_nr(jnp.broadcast_to(jnp.sum(tot), (16,)))
        inv = rt * rt
        for k in range(NLV):
            wbuf[i, pl.ds(k * 16, 16)] = evs[k] * inv

    def task(t_off, k, drain_prev_cond, fire_next_cond):
        h = t_off % 2
        i = 2 * k + t_off // 2
        ga, gp, gsem, osem = sets[t_off]
        nga, ngp, ngsem, nosem = sets[(t_off + 2) % 4]
        span = spans[h]
        _drain(amp_hbm.at[span], ga, gsem)
        _drain(amp_hbm.at[span], gp, gsem)
        do_half(ga, gp, i, h)
        if h == 1:
            softmax_row(i)
        def drain_prev():
            _drain(nga, real_hbm.at[0, span], nosem)
            _drain(ngp, imag_hbm.at[0, span], nosem)

        def fire_next():
            fire_gathers(i + 1, h, nga, ngp, ngsem)

        if drain_prev_cond is True:
            drain_prev()
        else:
            pl.when(drain_prev_cond)(drain_prev)
        if fire_next_cond is True:
            fire_next()
        else:
            pl.when(fire_next_cond)(fire_next)
        pltpu.async_copy(ga, real_hbm.at[base + i, span], osem)
        pltpu.async_copy(gp, imag_hbm.at[base + i, span], osem)

    fire_gathers(0, 0, ga0, gp0, gsem0)
    fire_gathers(0, 1, ga1, gp1, gsem1)

    def quad_body(k, c):
        task(0, k, k > 0, True)
        task(1, k, k > 0, True)
        task(2, k, True, k < ROWS_PER_W // 2 - 1)
        task(3, k, True, k < ROWS_PER_W // 2 - 1)
        return c
    lax.fori_loop(0, ROWS_PER_W // 2, quad_body, 0)

    _drain(ga2, real_hbm.at[0, spans[0]], osem2)
    _drain(gp2, imag_hbm.at[0, spans[0]], osem2)
    _drain(ga3, real_hbm.at[0, spans[1]], osem3)
    _drain(gp3, imag_hbm.at[0, spans[1]], osem3)
    pltpu.sync_copy(wbuf, w_hbm.at[pl.ds(base, ROWS_PER_W)])


_sc_call = pl.kernel(
    _body,
    out_type=(
        jax.ShapeDtypeStruct((B, L, DIM), jnp.float32),
        jax.ShapeDtypeStruct((B, L, DIM), jnp.float32),
        jax.ShapeDtypeStruct((B, LP), jnp.float32),
    ),
    mesh=plsc.VectorSubcoreMesh(core_axis_name="c", subcore_axis_name="s"),
    scratch_types=[
        pltpu.VMEM((ROWS_PER_W, 1, HA), jnp.int32),
        pltpu.VMEM((ROWS_PER_W, 1, HB), jnp.int32),
        pltpu.VMEM((HA, DIM), jnp.float32),
        pltpu.VMEM((HA, DIM), jnp.float32),
        pltpu.VMEM((HB, DIM), jnp.float32),
        pltpu.VMEM((HB, DIM), jnp.float32),
        pltpu.VMEM((HA, DIM), jnp.float32),
        pltpu.VMEM((HA, DIM), jnp.float32),
        pltpu.VMEM((HB, DIM), jnp.float32),
        pltpu.VMEM((HB, DIM), jnp.float32),
        pltpu.VMEM((ROWS_PER_W, LP), jnp.float32),
        pltpu.SemaphoreType.DMA,
        pltpu.SemaphoreType.DMA,
        pltpu.SemaphoreType.DMA,
        pltpu.SemaphoreType.DMA,
        pltpu.SemaphoreType.DMA,
        pltpu.SemaphoreType.DMA,
        pltpu.SemaphoreType.DMA,
        pltpu.SemaphoreType.DMA,
    ],
    compiler_params=pltpu.CompilerParams(needs_layout_passes=False),
)


def kernel(doc, amplitude_table, phase_table):
    doc_flat = doc.reshape(B * L).astype(jnp.int32)
    real, imag, w = _sc_call(doc_flat, amplitude_table, phase_table)
    return real, imag, w[:, :L]


# speedup vs baseline: 1.2647x; 1.1170x over previous
import jax
import jax.numpy as jnp
import numpy as np
from jax import lax
from jax.experimental import pallas as pl
from jax.experimental.pallas import tpu as pltpu
from jax.experimental.pallas import tpu_sc as plsc

VOCAB = 100000
DIM = 128
B = 1024
L = 200
LP = 256
NLV = 13
NC = 2
NS = 16
NW = NC * NS
ROWS_PER_W = B // NW
HA = 96
HB = 104
GA_N = HA // 16
GB_N = HB // 16

PI = np.float32(np.pi)
QS = tuple(np.float32(v) for v in
           (-0.9878619909286499, 0.15527130663394928, -0.005643106531351805))
RC = tuple(np.float32(v) for v in
           (-0.9989871382713318, 0.49624863266944885,
            -0.03952230140566826, 0.0009928615763783455))


def _poly(u, coeffs):
    acc = coeffs[-1]
    for c in reversed(coeffs[:-1]):
        acc = acc * u + c
    return acc


def _rsqrt_nr(s):
    i = plsc.bitcast(s, jnp.int32)
    i = jnp.int32(0x5F3759DF) - lax.shift_right_logical(i, 1)
    y = plsc.bitcast(i, jnp.float32)
    hs = s * np.float32(0.5)
    for _ in range(3):
        y = y * (np.float32(1.5) - hs * y * y)
    return y


def _drain(src, dst, sem):
    pltpu.make_async_copy(src, dst, sem).wait()


def _body(doc_hbm, amp_hbm, ph_hbm, real_hbm, imag_hbm, w_hbm,
          idx_a, idx_b, ga0, gp0, ga1, gp1, ga2, gp2, ga3, gp3, wbuf,
          gsem0, gsem1, gsem2, gsem3, osem0, osem1, osem2, osem3):
    wid = lax.axis_index("s") * NC + lax.axis_index("c")
    base = wid * ROWS_PER_W
    lane = lax.broadcasted_iota(jnp.int32, (16,), 0)

    sets = ((ga0, gp0, gsem0, osem0), (ga1, gp1, gsem1, osem1),
            (ga2, gp2, gsem2, osem2), (ga3, gp3, gsem3, osem3))
    spans = (pl.ds(0, HA), pl.ds(HA, HB))

    for i in range(ROWS_PER_W):
        r = (base + i) * L
        pltpu.async_copy(doc_hbm.at[pl.ds(r, HA)], idx_a.at[i, 0], osem0)
        pltpu.async_copy(doc_hbm.at[pl.ds(r + HA, HB)], idx_b.at[i, 0], osem0)
    for i in range(ROWS_PER_W):
        _drain(doc_hbm.at[pl.ds(0, HA)], idx_a.at[i, 0], osem0)
        _drain(doc_hbm.at[pl.ds(0, HB)], idx_b.at[i, 0], osem0)

    def fire_gathers(i, h, ga, gp, gsem):
        idx = (idx_a if h == 0 else idx_b).at[i, 0]
        pltpu.async_copy(amp_hbm.at[idx], ga, gsem)
        pltpu.async_copy(ph_hbm.at[idx], gp, gsem)

    QB = tuple(jnp.bfloat16(float(v)) for v in QS)
    RB = tuple(jnp.bfloat16(float(v)) for v in RC)

    def one_token(ga, gp, lt):
        acc = jnp.zeros((16,), jnp.float32)
        for j in range(DIM // 32):
            sl0 = pl.ds(j * 32, 16)
            sl1 = pl.ds(j * 32 + 16, 16)
            a0, a1 = ga[lt, sl0], ga[lt, sl1]
            p0, p1 = gp[lt, sl0], gp[lt, sl1]
            tt0, tt1 = p0 - PI, p1 - PI
            tb = plsc.pack(tt0, tt1, format=plsc.PackFormat.INTERLEAVED)
            ub = tb * tb
            rb = _poly(ub, RB)
            qb = _poly(ub, QB)
            r0, r1 = plsc.unpack(rb, format=plsc.PackFormat.INTERLEAVED)
            q0, q1 = plsc.unpack(qb, format=plsc.PackFormat.INTERLEAVED)
            ga[lt, sl0] = a0 * r0
            ga[lt, sl1] = a1 * r1
            gp[lt, sl0] = (a0 * tt0) * q0
            gp[lt, sl1] = (a1 * tt1) * q1
            acc = acc + a0 * a0
            acc = acc + a1 * a1
        return jnp.sum(acc)

    def do_half(ga, gp, i, h):
        goff = h * GA_N

        def group(lt0, n_tok, gidx, init):
            @plsc.parallel_loop(0, n_tok, unroll=8, carry=init)
            def g(ti, gacc):
                return jnp.where(lane == ti, one_token(ga, gp, lt0 + ti), gacc)
            wbuf[i, pl.ds(gidx * 16, 16)] = g

        @plsc.parallel_loop(0, GA_N if h == 0 else GB_N)
        def _(lg):
            group(lg * 16, 16, goff + lg, jnp.zeros((16,), jnp.float32))
        if h == 1:
            group(GB_N * 16, HB - GB_N * 16, goff + GB_N,
                  jnp.full((16,), 1e-30, jnp.float32))

    def softmax_row(i):
        svs = [wbuf[i, pl.ds(k * 16, 16)] for k in range(NLV)]
        nvs = [s * _rsqrt_nr(s) for s in svs]
        m = nvs[0]
        for v in nvs[1:]:
            m = jnp.maximum(m, v)
        mm = jnp.max(m)
        evs = [jnp.exp(v - mm) for v in nvs]
        tot = evs[0]
        for v in evs[1:]:
            tot = tot + v
        rt = _rsqrt_nr(jnp.broadcast_to(jnp.sum(tot), (16,)))
        inv = rt * rt
        for k in range(NLV):
            wbuf[i, pl.ds(k * 16, 16)] = evs[k] * inv

    def task(t_off, k, drain_prev_cond, fire_next_cond):
        h = t_off % 2
        i = 2 * k + t_off // 2
        ga, gp, gsem, osem = sets[t_off]
        nga, ngp, ngsem, nosem = sets[(t_off + 2) % 4]
        span = spans[h]
        _drain(amp_hbm.at[span], ga, gsem)
        _drain(amp_hbm.at[span], gp, gsem)
        do_half(ga, gp, i, h)
        if h == 1:
            softmax_row(i)
        def drain_prev():
            _drain(nga, real_hbm.at[0, span], nosem)
            _drain(ngp, imag_hbm.at[0, span], nosem)

        def fire_next():
            fire_gathers(i + 1, h, nga, ngp, ngsem)

        if drain_prev_cond is True:
            drain_prev()
        else:
            pl.when(drain_prev_cond)(drain_prev)
        if fire_next_cond is True:
            fire_next()
        else:
            pl.when(fire_next_cond)(fire_next)
        pltpu.async_copy(ga, real_hbm.at[base + i, span], osem)
        pltpu.async_copy(gp, imag_hbm.at[base + i, span], osem)

    fire_gathers(0, 0, ga0, gp0, gsem0)
    fire_gathers(0, 1, ga1, gp1, gsem1)

    def quad_body(k, c):
        task(0, k, k > 0, True)
        task(1, k, k > 0, True)
        task(2, k, True, k < ROWS_PER_W // 2 - 1)
        task(3, k, True, k < ROWS_PER_W // 2 - 1)
        return c
    lax.fori_loop(0, ROWS_PER_W // 2, quad_body, 0)

    _drain(ga2, real_hbm.at[0, spans[0]], osem2)
    _drain(gp2, imag_hbm.at[0, spans[0]], osem2)
    _drain(ga3, real_hbm.at[0, spans[1]], osem3)
    _drain(gp3, imag_hbm.at[0, spans[1]], osem3)
    pltpu.sync_copy(wbuf, w_hbm.at[pl.ds(base, ROWS_PER_W)])


_sc_call = pl.kernel(
    _body,
    out_type=(
        jax.ShapeDtypeStruct((B, L, DIM), jnp.float32),
        jax.ShapeDtypeStruct((B, L, DIM), jnp.float32),
        jax.ShapeDtypeStruct((B, LP), jnp.float32),
    ),
    mesh=plsc.VectorSubcoreMesh(core_axis_name="c", subcore_axis_name="s"),
    scratch_types=[
        pltpu.VMEM((ROWS_PER_W, 1, HA), jnp.int32),
        pltpu.VMEM((ROWS_PER_W, 1, HB), jnp.int32),
        pltpu.VMEM((HA, DIM), jnp.float32),
        pltpu.VMEM((HA, DIM), jnp.float32),
        pltpu.VMEM((HB, DIM), jnp.float32),
        pltpu.VMEM((HB, DIM), jnp.float32),
        pltpu.VMEM((HA, DIM), jnp.float32),
        pltpu.VMEM((HA, DIM), jnp.float32),
        pltpu.VMEM((HB, DIM), jnp.float32),
        pltpu.VMEM((HB, DIM), jnp.float32),
        pltpu.VMEM((ROWS_PER_W, LP), jnp.float32),
        pltpu.SemaphoreType.DMA,
        pltpu.SemaphoreType.DMA,
        pltpu.SemaphoreType.DMA,
        pltpu.SemaphoreType.DMA,
        pltpu.SemaphoreType.DMA,
        pltpu.SemaphoreType.DMA,
        pltpu.SemaphoreType.DMA,
        pltpu.SemaphoreType.DMA,
    ],
    compiler_params=pltpu.CompilerParams(needs_layout_passes=False),
)


def kernel(doc, amplitude_table, phase_table):
    doc_flat = doc.reshape(B * L).astype(jnp.int32)
    real, imag, w = _sc_call(doc_flat, amplitude_table, phase_table)
    return real, imag, w[:, :L]
